# baseline (speedup 1.0000x reference)
import functools

import jax
import jax.numpy as jnp
from jax import lax
from jax.experimental import pallas as pl
from jax.experimental.pallas import tpu as pltpu
from jax.experimental.pallas import tpu_sc as plsc

VOCAB = 100000
EMBED = 16
BATCH = 1024


_info = plsc.get_sparse_core_info()
_NC, _NS = _info.num_cores, _info.num_subcores
_NW = _NC * _NS
_B_PER_W = BATCH // _NW
_LINES = VOCAB // 8

_sc_mesh = plsc.VectorSubcoreMesh(core_axis_name="c", subcore_axis_name="s")


@functools.partial(
    pl.kernel,
    mesh=_sc_mesh,
    out_type=jax.ShapeDtypeStruct((BATCH, 128), jnp.float32),
    scratch_types=[
        pltpu.VMEM((_B_PER_W,), jnp.int32),
        pltpu.VMEM((_B_PER_W,), jnp.int32),
        pltpu.VMEM((_B_PER_W, 128), jnp.float32),
        pltpu.SemaphoreType.DMA,
    ],
)
def _sc_gather(table_hbm, idx_hbm, out_hbm, idx_v, line_v, rows_v, sem):
    wid = lax.axis_index("s") * _NC + lax.axis_index("c")
    base = wid * _B_PER_W
    pltpu.sync_copy(idx_hbm.at[pl.ds(base, _B_PER_W)], idx_v)
    for c in range(_B_PER_W // 16):
        line_v[pl.ds(c * 16, 16)] = lax.shift_right_logical(
            idx_v[pl.ds(c * 16, 16)], 3)
    pltpu.async_copy(table_hbm.at[line_v], rows_v, sem).wait()
    pltpu.sync_copy(rows_v, out_hbm.at[pl.ds(base, _B_PER_W)])


_VT = 2048


def _proj_body(idx_ref, x128_ref, w_ref, b_ref, out_ref, x_scr):
    @pl.when(pl.program_id(0) == 0)
    def _extract():
        off = idx_ref[...] & 7
        x128 = x128_ref[...]
        acc = x128[:, 0:EMBED]
        for o in range(1, 8):
            acc = jnp.where(off == o, x128[:, o * EMBED:(o + 1) * EMBED], acc)
        x_scr[...] = acc
    out_ref[...] = lax.dot_general(
        x_scr[...], w_ref[...],
        dimension_numbers=(((1,), (1,)), ((), ())),
        preferred_element_type=jnp.float32,
    ) + b_ref[...]


def _projection(idx2d, x128, lin_w, lin_b2d):
    grid = pl.cdiv(VOCAB, _VT)
    return pl.pallas_call(
        _proj_body,
        grid=(grid,),
        in_specs=[
            pl.BlockSpec((BATCH, 1), lambda i: (0, 0)),
            pl.BlockSpec((BATCH, 128), lambda i: (0, 0)),
            pl.BlockSpec((_VT, EMBED), lambda i: (i, 0)),
            pl.BlockSpec((1, _VT), lambda i: (0, i)),
        ],
        out_specs=pl.BlockSpec((BATCH, _VT), lambda i: (0, i)),
        out_shape=jax.ShapeDtypeStruct((BATCH, VOCAB), jnp.float32),
        scratch_shapes=[pltpu.VMEM((BATCH, EMBED), jnp.float32)],
    )(idx2d, x128, lin_w, lin_b2d)


def kernel(inputs_, emb_table, lin_w, lin_b):
    idx = inputs_.astype(jnp.int32)
    table_lines = emb_table.reshape(_LINES, 128)
    x128 = _sc_gather(table_lines, idx)
    return _projection(idx.reshape(BATCH, 1), x128, lin_w,
                       lin_b.reshape(1, VOCAB))


# speedup vs baseline: 1.0774x; 1.0774x over previous
import functools

import jax
import jax.numpy as jnp
from jax import lax
from jax.experimental import pallas as pl
from jax.experimental.pallas import tpu as pltpu
from jax.experimental.pallas import tpu_sc as plsc

VOCAB = 100000
EMBED = 16
BATCH = 1024


_info = plsc.get_sparse_core_info()
_NC, _NS = _info.num_cores, _info.num_subcores
_NW = _NC * _NS
_B_PER_W = BATCH // _NW
_LINES = VOCAB // 8

_sc_mesh = plsc.VectorSubcoreMesh(core_axis_name="c", subcore_axis_name="s")


@functools.partial(
    pl.kernel,
    mesh=_sc_mesh,
    out_type=jax.ShapeDtypeStruct((BATCH, 128), jnp.float32),
    scratch_types=[
        pltpu.VMEM((_B_PER_W,), jnp.int32),
        pltpu.VMEM((_B_PER_W,), jnp.int32),
        pltpu.VMEM((_B_PER_W, 128), jnp.float32),
        pltpu.SemaphoreType.DMA,
    ],
)
def _sc_gather(table_hbm, idx_hbm, out_hbm, idx_v, line_v, rows_v, sem):
    wid = lax.axis_index("s") * _NC + lax.axis_index("c")
    base = wid * _B_PER_W
    pltpu.sync_copy(idx_hbm.at[pl.ds(base, _B_PER_W)], idx_v)
    for c in range(_B_PER_W // 16):
        line_v[pl.ds(c * 16, 16)] = lax.shift_right_logical(
            idx_v[pl.ds(c * 16, 16)], 3)
    pltpu.async_copy(table_hbm.at[line_v], rows_v, sem).wait()
    pltpu.sync_copy(rows_v, out_hbm.at[pl.ds(base, _B_PER_W)])


_VT = 2048


def _proj_body(idx_ref, x128_ref, w_ref, b_ref, out_ref, x_scr):
    @pl.when(pl.program_id(0) == 0)
    def _extract():
        off = idx_ref[...] & 7
        x128 = x128_ref[...]
        acc = x128[:, 0:EMBED]
        for o in range(1, 8):
            acc = jnp.where(off == o, x128[:, o * EMBED:(o + 1) * EMBED], acc)
        x_scr[...] = acc
    out_ref[...] = jnp.dot(
        x_scr[...], w_ref[...], preferred_element_type=jnp.float32,
    ) + b_ref[...]


def _projection(idx2d, x128, lin_wT, lin_b2d):
    grid = pl.cdiv(VOCAB, _VT)
    return pl.pallas_call(
        _proj_body,
        grid=(grid,),
        in_specs=[
            pl.BlockSpec((BATCH, 1), lambda i: (0, 0)),
            pl.BlockSpec((BATCH, 128), lambda i: (0, 0)),
            pl.BlockSpec((EMBED, _VT), lambda i: (0, i)),
            pl.BlockSpec((1, _VT), lambda i: (0, i)),
        ],
        out_specs=pl.BlockSpec((BATCH, _VT), lambda i: (0, i)),
        out_shape=jax.ShapeDtypeStruct((BATCH, VOCAB), jnp.float32),
        scratch_shapes=[pltpu.VMEM((BATCH, EMBED), jnp.float32)],
    )(idx2d, x128, lin_wT, lin_b2d)


def kernel(inputs_, emb_table, lin_w, lin_b):
    idx = inputs_.astype(jnp.int32)
    table_lines = emb_table.reshape(_LINES, 128)
    x128 = _sc_gather(table_lines, idx)
    return _projection(idx.reshape(BATCH, 1), x128, lin_w.T,
                       lin_b.reshape(1, VOCAB))
